# baseline (speedup 1.0000x reference)
import jax
import jax.numpy as jnp
from jax.experimental import pallas as pl


def _bf16_dot(a, b):
    return jnp.dot(a.astype(jnp.bfloat16), b.astype(jnp.bfloat16),
                   preferred_element_type=jnp.float32)


def _mm_kernel(x_ref, w_ref, o_ref):
    o_ref[...] = _bf16_dot(x_ref[...], w_ref[...])


def _layer1_kernel(adj_ref, s1_ref, b1_ref, w2_ref, o_ref):
    acc = _bf16_dot(adj_ref[...], s1_ref[...])
    h = jnp.maximum(acc + b1_ref[...], 0.0)
    o_ref[...] = _bf16_dot(h, w2_ref[...])


def _layer2_kernel(adj_ref, s2_ref, b2_ref, o_ref):
    acc = _bf16_dot(adj_ref[...], s2_ref[...])
    o_ref[...] = acc + b2_ref[...]


def kernel(x, adj, W1, b1, W2, b2):
    n, d_in = x.shape
    hidden = W1.shape[1]
    ncls = W2.shape[1]

    bm_a = 1000
    s1 = pl.pallas_call(
        _mm_kernel,
        grid=(n // bm_a,),
        in_specs=[
            pl.BlockSpec((bm_a, d_in), lambda i: (i, 0)),
            pl.BlockSpec((d_in, hidden), lambda i: (0, 0)),
        ],
        out_specs=pl.BlockSpec((bm_a, hidden), lambda i: (i, 0)),
        out_shape=jax.ShapeDtypeStruct((n, hidden), jnp.float32),
    )(x, W1)

    bm = 400
    s2 = pl.pallas_call(
        _layer1_kernel,
        grid=(n // bm,),
        in_specs=[
            pl.BlockSpec((bm, n), lambda i: (i, 0)),
            pl.BlockSpec((n, hidden), lambda i: (0, 0)),
            pl.BlockSpec((1, hidden), lambda i: (0, 0)),
            pl.BlockSpec((hidden, ncls), lambda i: (0, 0)),
        ],
        out_specs=pl.BlockSpec((bm, ncls), lambda i: (i, 0)),
        out_shape=jax.ShapeDtypeStruct((n, ncls), jnp.float32),
    )(adj, s1, b1.reshape(1, hidden), W2)

    out = pl.pallas_call(
        _layer2_kernel,
        grid=(n // bm,),
        in_specs=[
            pl.BlockSpec((bm, n), lambda i: (i, 0)),
            pl.BlockSpec((n, ncls), lambda i: (0, 0)),
            pl.BlockSpec((1, ncls), lambda i: (0, 0)),
        ],
        out_specs=pl.BlockSpec((bm, ncls), lambda i: (i, 0)),
        out_shape=jax.ShapeDtypeStruct((n, ncls), jnp.float32),
    )(adj, s2, b2.reshape(1, ncls))
    return out


# speedup vs baseline: 1.0632x; 1.0632x over previous
import jax
import jax.numpy as jnp
from jax.experimental import pallas as pl
from jax.experimental.pallas import tpu as pltpu

_BM = 400


def _bf16_dot(a, b):
    return jnp.dot(a.astype(jnp.bfloat16), b.astype(jnp.bfloat16),
                   preferred_element_type=jnp.float32)


def _fused_kernel(x_ref, adj_ref, w1_ref, b1_ref, w2_ref, b2_ref,
                  o_ref, s1_ref, s2_ref):
    p = pl.program_id(0)
    i = pl.program_id(1)

    @pl.when((p == 0) & (i == 0))
    def _init_s1():
        s1_ref[...] = _bf16_dot(x_ref[...], w1_ref[...])

    @pl.when(p == 0)
    def _phase0():
        acc = _bf16_dot(adj_ref[...], s1_ref[...])
        h = jnp.maximum(acc + b1_ref[...], 0.0)
        s2_ref[pl.ds(i * _BM, _BM), :] = _bf16_dot(h, w2_ref[...])
        o_ref[...] = jnp.zeros_like(o_ref)

    @pl.when(p == 1)
    def _phase1():
        acc = _bf16_dot(adj_ref[...], s2_ref[...])
        o_ref[...] = acc + b2_ref[...]


def kernel(x, adj, W1, b1, W2, b2):
    n, d_in = x.shape
    hidden = W1.shape[1]
    ncls = W2.shape[1]

    return pl.pallas_call(
        _fused_kernel,
        grid=(2, n // _BM),
        in_specs=[
            pl.BlockSpec((n, d_in), lambda p, i: (0, 0)),
            pl.BlockSpec((_BM, n), lambda p, i: (i, 0)),
            pl.BlockSpec((d_in, hidden), lambda p, i: (0, 0)),
            pl.BlockSpec((1, hidden), lambda p, i: (0, 0)),
            pl.BlockSpec((hidden, ncls), lambda p, i: (0, 0)),
            pl.BlockSpec((1, ncls), lambda p, i: (0, 0)),
        ],
        out_specs=pl.BlockSpec((_BM, ncls), lambda p, i: (i, 0)),
        out_shape=jax.ShapeDtypeStruct((n, ncls), jnp.float32),
        scratch_shapes=[
            pltpu.VMEM((n, hidden), jnp.float32),
            pltpu.VMEM((n, ncls), jnp.float32),
        ],
    )(x, adj, W1, b1.reshape(1, hidden), W2, b2.reshape(1, ncls))
